# Initial kernel scaffold; baseline (speedup 1.0000x reference)
import jax
import jax.numpy as jnp
from jax.experimental import pallas as pl


def kernel(B, A, atom_positions, block_id, block_table, atom_table, pos_table):
    raise NotImplementedError("write your pallas kernel here")


# speedup vs baseline: 5.6942x; 5.6942x over previous
import functools

import jax
import jax.numpy as jnp
from jax import lax
from jax.experimental import pallas as pl
from jax.experimental.pallas import tpu as pltpu
from jax.experimental.pallas import tpu_sc as plsc

_NBT = 32
_NAT = 119
_NAP = 16
_E = 64
_NB = 50000
_NU = 400000
_NW = 32
_CHUNK = 256
_SUB = 128
_NUP = 409600
_NCHUNKS = _NUP // _CHUNK
_CPW = _NCHUNKS // _NW
_FR = _NAT * _NAP * _NBT


def _build_body(atab_ref, ptab_ref, btab_ref, out_ref):
    a = atab_ref[...]
    p = ptab_ref[...]
    b = btab_ref[...]
    ap = a[:, None, :] + p[None, :, :]
    apb = ap[:, :, None, :] + b[None, None, :, :]
    out_ref[...] = apb.reshape(_FR, _E)


_build_fused = pl.pallas_call(
    _build_body,
    out_shape=jax.ShapeDtypeStruct((_FR, _E), jnp.float32),
)


def _sc_body(fused, btab, ai, pi, bi, out, b_v, a_v, p_v, i_v, idx2, acc, sem):
    wid = lax.axis_index("s") * 2 + lax.axis_index("c")
    pltpu.sync_copy(btab, b_v)

    def chunk(j, carry):
        base = (j * _NW + wid) * _CHUNK
        pltpu.sync_copy(ai.at[pl.ds(base, _CHUNK)], a_v)
        pltpu.sync_copy(pi.at[pl.ds(base, _CHUNK)], p_v)
        pltpu.sync_copy(bi.at[pl.ds(base, _CHUNK)], i_v)
        for i in range(_CHUNK // 16):
            s = pl.ds(i * 16, 16)
            bt = plsc.load_gather(b_v, [i_v[s]])
            idx2[i // 8, pl.ds((i % 8) * 16, 16)] = a_v[s] * 512 + p_v[s] * 32 + bt
        cps = [
            pltpu.async_copy(fused.at[idx2.at[k]], acc.at[pl.ds(k * _SUB, _SUB)], sem)
            for k in range(_CHUNK // _SUB)
        ]
        for cp in cps:
            cp.wait()
        pltpu.sync_copy(acc, out.at[pl.ds(base, _CHUNK)])
        return carry

    lax.fori_loop(0, _CPW, chunk, 0)


_sc_gather = functools.partial(
    pl.kernel,
    out_type=jax.ShapeDtypeStruct((_NUP, _E), jnp.float32),
    mesh=plsc.VectorSubcoreMesh(core_axis_name="c", subcore_axis_name="s"),
    scratch_types=[
        pltpu.VMEM((_NB,), jnp.int32),
        pltpu.VMEM((_CHUNK,), jnp.int32),
        pltpu.VMEM((_CHUNK,), jnp.int32),
        pltpu.VMEM((_CHUNK,), jnp.int32),
        pltpu.VMEM((_CHUNK // _SUB, _SUB), jnp.int32),
        pltpu.VMEM((_CHUNK, _E), jnp.float32),
        pltpu.SemaphoreType.DMA,
    ],
    compiler_params=pltpu.CompilerParams(
        needs_layout_passes=False, use_tc_tiling_on_sc=False
    ),
)(_sc_body)


def kernel(B, A, atom_positions, block_id, block_table, atom_table, pos_table):
    b32 = B.astype(jnp.int32)
    pad = _NUP - _NU
    a32 = jnp.pad(A.astype(jnp.int32), (0, pad))
    p32 = jnp.pad(atom_positions.astype(jnp.int32), (0, pad))
    i32 = jnp.pad(block_id.astype(jnp.int32), (0, pad))
    fused = _build_fused(atom_table, pos_table, block_table)
    out = _sc_gather(fused, b32, a32, p32, i32)
    return out[:_NU]


# speedup vs baseline: 5.8748x; 1.0317x over previous
import functools

import jax
import jax.numpy as jnp
from jax import lax
from jax.experimental import pallas as pl
from jax.experimental.pallas import tpu as pltpu
from jax.experimental.pallas import tpu_sc as plsc

_NBT = 32
_NAT = 119
_NAP = 16
_E = 64
_NB = 50000
_NU = 400000
_NW = 32
_CHUNK = 256
_SUB = 128
_NUP = 409600
_NCHUNKS = _NUP // _CHUNK
_CPW = _NCHUNKS // _NW
_FR = _NAT * _NAP * _NBT


def _build_body(atab_ref, ptab_ref, btab_ref, out_ref):
    a = atab_ref[...]
    p = ptab_ref[...]
    b = btab_ref[...]
    ap = a[:, None, :] + p[None, :, :]
    apb = ap[:, :, None, :] + b[None, None, :, :]
    out_ref[...] = apb.reshape(_FR, _E)


_build_fused = pl.pallas_call(
    _build_body,
    out_shape=jax.ShapeDtypeStruct((_FR, _E), jnp.float32),
)


_SPAN = _NUP // _NW


def _sc_body(
    fused, btab, ai, pi, bi, out,
    b_v, a_v, p_v, i_v, idx0, idx1, acc0, acc1, s_in, g0, g1, o0, o1,
):
    wid = lax.axis_index("s") * 2 + lax.axis_index("c")
    span = wid * _SPAN
    pltpu.sync_copy(btab, b_v)
    for src, dst in ((ai, a_v), (pi, p_v), (bi, i_v)):
        pltpu.async_copy(src.at[pl.ds(span, _SPAN)], dst, s_in)
    for src, dst in ((ai, a_v), (pi, p_v), (bi, i_v)):
        pltpu.make_async_copy(src.at[pl.ds(span, _SPAN)], dst, s_in).wait()

    def compute_idx(jj, idx2):
        for i in range(_CHUNK // 16):
            s = pl.ds(jj * _CHUNK + i * 16, 16)
            bt = plsc.load_gather(b_v, [i_v[s]])
            idx2[i // 8, pl.ds((i % 8) * 16, 16)] = a_v[s] * 512 + p_v[s] * 32 + bt

    def start_gathers(idx2, acc, sem):
        for k in range(_CHUNK // _SUB):
            pltpu.async_copy(fused.at[idx2.at[k]], acc.at[pl.ds(k * _SUB, _SUB)], sem)

    def wait_gathers(idx2, acc, sem):
        for k in range(_CHUNK // _SUB):
            pltpu.make_async_copy(
                fused.at[idx2.at[k]], acc.at[pl.ds(k * _SUB, _SUB)], sem
            ).wait()

    def start_out(c, acc, sem):
        pltpu.async_copy(acc, out.at[pl.ds(span + c * _CHUNK, _CHUNK)], sem)

    def drain_out(c, acc, sem):
        pltpu.make_async_copy(
            acc, out.at[pl.ds(span + c * _CHUNK, _CHUNK)], sem
        ).wait()

    def pair(t, carry):
        c0 = 2 * t
        c1 = c0 + 1
        compute_idx(c0, idx0)

        @pl.when(t > 0)
        def _():
            drain_out(c0 - 2, acc0, o0)

        start_gathers(idx0, acc0, g0)
        compute_idx(c1, idx1)

        @pl.when(t > 0)
        def _():
            drain_out(c1 - 2, acc1, o1)

        start_gathers(idx1, acc1, g1)
        wait_gathers(idx0, acc0, g0)
        start_out(c0, acc0, o0)
        wait_gathers(idx1, acc1, g1)
        start_out(c1, acc1, o1)
        return carry

    lax.fori_loop(0, _CPW // 2, pair, 0)
    drain_out(_CPW - 2, acc0, o0)
    drain_out(_CPW - 1, acc1, o1)


_sc_gather = functools.partial(
    pl.kernel,
    out_type=jax.ShapeDtypeStruct((_NUP, _E), jnp.float32),
    mesh=plsc.VectorSubcoreMesh(core_axis_name="c", subcore_axis_name="s"),
    scratch_types=[
        pltpu.VMEM((_NB,), jnp.int32),
        pltpu.VMEM((_SPAN,), jnp.int32),
        pltpu.VMEM((_SPAN,), jnp.int32),
        pltpu.VMEM((_SPAN,), jnp.int32),
        pltpu.VMEM((_CHUNK // _SUB, _SUB), jnp.int32),
        pltpu.VMEM((_CHUNK // _SUB, _SUB), jnp.int32),
        pltpu.VMEM((_CHUNK, _E), jnp.float32),
        pltpu.VMEM((_CHUNK, _E), jnp.float32),
        pltpu.SemaphoreType.DMA,
        pltpu.SemaphoreType.DMA,
        pltpu.SemaphoreType.DMA,
        pltpu.SemaphoreType.DMA,
        pltpu.SemaphoreType.DMA,
    ],
    compiler_params=pltpu.CompilerParams(
        needs_layout_passes=False, use_tc_tiling_on_sc=False
    ),
)(_sc_body)


def kernel(B, A, atom_positions, block_id, block_table, atom_table, pos_table):
    b32 = B.astype(jnp.int32)
    pad = _NUP - _NU
    a32 = jnp.pad(A.astype(jnp.int32), (0, pad))
    p32 = jnp.pad(atom_positions.astype(jnp.int32), (0, pad))
    i32 = jnp.pad(block_id.astype(jnp.int32), (0, pad))
    fused = _build_fused(atom_table, pos_table, block_table)
    out = _sc_gather(fused, b32, a32, p32, i32)
    return out[:_NU]


# speedup vs baseline: 11.8226x; 2.0124x over previous
import functools

import jax
import jax.numpy as jnp
from jax import lax
from jax.experimental import pallas as pl
from jax.experimental.pallas import tpu as pltpu
from jax.experimental.pallas import tpu_sc as plsc

_NBT = 32
_NAT = 119
_NAP = 16
_E = 64
_NB = 50000
_NU = 400000
_NW = 32
_SUB = 128
_FR = _NAT * _NAP * _NBT
_SPAN = _NU // _NW
_CHUNK = 250
_GCHUNK = 256
_CPW = _SPAN // _CHUNK
_LOAD = 12512
_NPAD = 400064
_ORPW = _SPAN // 2
_ORC = _CHUNK // 2


def _build_body(atab_ref, ptab_ref, btab_ref, out_ref):
    a = atab_ref[...]
    p = ptab_ref[...]
    b = btab_ref[...]
    ap = a[:, None, :] + p[None, :, :]
    apb = ap[:, :, None, :] + b[None, None, :, :]
    out_ref[...] = apb.reshape(_FR, _E)


_build_fused = pl.pallas_call(
    _build_body,
    out_shape=jax.ShapeDtypeStruct((_FR, _E), jnp.float32),
)


def _sc_body(
    fused, btab, ai, pi, bi, out,
    b_v, a_v, p_v, i_v, idx0, idx1, acc0, acc1, s_in, g0, g1, o0, o1,
):
    wid = lax.axis_index("s") * 2 + lax.axis_index("c")
    off = (wid % 2) * 4
    base = pl.multiple_of(wid * _SPAN - off, 8)
    pltpu.sync_copy(btab, b_v)
    for src, dst in ((ai, a_v), (pi, p_v), (bi, i_v)):
        pltpu.async_copy(src.at[pl.ds(base, _LOAD)], dst, s_in)
    for src, dst in ((ai, a_v), (pi, p_v), (bi, i_v)):
        pltpu.make_async_copy(src.at[pl.ds(base, _LOAD)], dst, s_in).wait()

    lanes = lax.iota(jnp.int32, 16)

    def compute_idx(jj, idx2):
        for i in range(_GCHUNK // 16):
            s = off + jj * _CHUNK + i * 16 + lanes
            a = plsc.load_gather(a_v, [s])
            p = plsc.load_gather(p_v, [s])
            ib = plsc.load_gather(i_v, [s])
            bt = plsc.load_gather(b_v, [ib])
            idx2[i // 8, pl.ds((i % 8) * 16, 16)] = a * 512 + p * 32 + bt

    def start_gathers(idx2, acc, sem):
        for k in range(_GCHUNK // _SUB):
            pltpu.async_copy(fused.at[idx2.at[k]], acc.at[pl.ds(k * _SUB, _SUB)], sem)

    def wait_gathers(idx2, acc, sem):
        for k in range(_GCHUNK // _SUB):
            pltpu.make_async_copy(
                fused.at[idx2.at[k]], acc.at[pl.ds(k * _SUB, _SUB)], sem
            ).wait()

    def start_out(c, acc, sem):
        pltpu.async_copy(
            acc.at[pl.ds(0, _CHUNK)],
            out.at[pl.ds(wid * _SPAN + c * _CHUNK, _CHUNK)],
            sem,
        )

    def drain_out(c, acc, sem):
        pltpu.make_async_copy(
            acc.at[pl.ds(0, _CHUNK)],
            out.at[pl.ds(wid * _SPAN + c * _CHUNK, _CHUNK)],
            sem,
        ).wait()

    def pair(t, carry):
        c0 = 2 * t
        c1 = c0 + 1
        compute_idx(c0, idx0)

        @pl.when(t > 0)
        def _():
            drain_out(c0 - 2, acc0, o0)

        start_gathers(idx0, acc0, g0)
        compute_idx(c1, idx1)

        @pl.when(t > 0)
        def _():
            drain_out(c1 - 2, acc1, o1)

        start_gathers(idx1, acc1, g1)
        wait_gathers(idx0, acc0, g0)
        start_out(c0, acc0, o0)
        wait_gathers(idx1, acc1, g1)
        start_out(c1, acc1, o1)
        return carry

    lax.fori_loop(0, _CPW // 2, pair, 0)
    drain_out(_CPW - 2, acc0, o0)
    drain_out(_CPW - 1, acc1, o1)


_sc_gather = functools.partial(
    pl.kernel,
    out_type=jax.ShapeDtypeStruct((_NU, _E), jnp.float32),
    mesh=plsc.VectorSubcoreMesh(core_axis_name="c", subcore_axis_name="s"),
    scratch_types=[
        pltpu.VMEM((_NB,), jnp.int32),
        pltpu.VMEM((_LOAD,), jnp.int32),
        pltpu.VMEM((_LOAD,), jnp.int32),
        pltpu.VMEM((_LOAD,), jnp.int32),
        pltpu.VMEM((_GCHUNK // _SUB, _SUB), jnp.int32),
        pltpu.VMEM((_GCHUNK // _SUB, _SUB), jnp.int32),
        pltpu.VMEM((_GCHUNK, _E), jnp.float32),
        pltpu.VMEM((_GCHUNK, _E), jnp.float32),
        pltpu.SemaphoreType.DMA,
        pltpu.SemaphoreType.DMA,
        pltpu.SemaphoreType.DMA,
        pltpu.SemaphoreType.DMA,
        pltpu.SemaphoreType.DMA,
    ],
    compiler_params=pltpu.CompilerParams(
        needs_layout_passes=False, use_tc_tiling_on_sc=False
    ),
)(_sc_body)


def kernel(B, A, atom_positions, block_id, block_table, atom_table, pos_table):
    b32 = B.astype(jnp.int32)
    pad = _NPAD - _NU
    a32 = jnp.pad(A.astype(jnp.int32), (0, pad))
    p32 = jnp.pad(atom_positions.astype(jnp.int32), (0, pad))
    i32 = jnp.pad(block_id.astype(jnp.int32), (0, pad))
    fused = _build_fused(atom_table, pos_table, block_table)
    return _sc_gather(fused, b32, a32, p32, i32)
